# Initial kernel scaffold; baseline (speedup 1.0000x reference)
import jax
import jax.numpy as jnp
from jax.experimental import pallas as pl


def kernel(x, tok_table, pos_table, gamma, beta):
    raise NotImplementedError("write your pallas kernel here")


# speedup vs baseline: 6.0426x; 6.0426x over previous
import functools

import jax
import jax.numpy as jnp
from jax import lax
from jax.experimental import pallas as pl
from jax.experimental.pallas import tpu as pltpu
from jax.experimental.pallas import tpu_sc as plsc

VOCAB = 4
SEQ = 10
D = 64
BATCH = 16384
ROWS = BATCH * SEQ
PAIRS = ROWS // 2
PSEQ = SEQ // 2

_info = plsc.get_sparse_core_info()
_NC = _info.num_cores
_NS = _info.num_subcores
_NW = _NC * _NS

PAIRS_PER_W = PAIRS // _NW
CHUNK = 320
N_CHUNKS = PAIRS_PER_W // CHUNK


def _lut_body(tok_ref, pos_ref, gamma_ref, beta_ref, lut_ref):
    tok = tok_ref[:, :]
    pos = pos_ref[:, :]
    e = tok[:, None, :] + pos[None, :, :]
    mean = jnp.mean(e, axis=-1, keepdims=True)
    c = e - mean
    var = jnp.mean(c * c, axis=-1, keepdims=True)
    normed = c * lax.rsqrt(var + 1e-5)
    lut_ref[:, :, :] = (normed * gamma_ref[0][None, None, :]
                        + beta_ref[0][None, None, :])


_lut_call = pl.pallas_call(
    _lut_body,
    out_shape=jax.ShapeDtypeStruct((VOCAB, SEQ, D), jnp.float32),
)


_sc_mesh = plsc.VectorSubcoreMesh(core_axis_name="c", subcore_axis_name="s")


@functools.partial(
    pl.kernel,
    mesh=_sc_mesh,
    out_type=jax.ShapeDtypeStruct((PAIRS, 2 * D), jnp.float32),
    scratch_types=[
        pltpu.VMEM((CHUNK,), jnp.int32),
        pltpu.VMEM((CHUNK,), jnp.int32),
        pltpu.VMEM((CHUNK,), jnp.int32),
        pltpu.VMEM((CHUNK, 2 * D), jnp.float32),
        pltpu.SemaphoreType.DMA,
    ],
)
def _sc_gather(xe_hbm, xo_hbm, plut_hbm, out_hbm, xe_v, xo_v, idx_v, rows_v,
               sem):
    wid = lax.axis_index("s") * _NC + lax.axis_index("c")
    base_w = wid * PAIRS_PER_W
    lane = lax.iota(jnp.int32, 16)

    def chunk_body(ci, carry):
        base = base_w + ci * CHUNK
        pltpu.sync_copy(xe_hbm.at[pl.ds(base, CHUNK)], xe_v)
        pltpu.sync_copy(xo_hbm.at[pl.ds(base, CHUNK)], xo_v)
        for g in range(CHUNK // 16):
            off = g * 16
            xe = xe_v[pl.ds(off, 16)]
            xo = xo_v[pl.ds(off, 16)]
            p = lax.rem(base + off + lane, PSEQ)
            idx_v[pl.ds(off, 16)] = (xe * VOCAB + xo) * PSEQ + p
        pltpu.async_copy(plut_hbm.at[idx_v], rows_v, sem).wait()
        pltpu.sync_copy(rows_v, out_hbm.at[pl.ds(base, CHUNK)])
        return carry

    lax.fori_loop(0, N_CHUNKS, chunk_body, 0)


def kernel(x, tok_table, pos_table, gamma, beta):
    lut = _lut_call(tok_table, pos_table,
                    gamma.reshape(1, D), beta.reshape(1, D))
    even = lut[:, 0::2, :]
    odd = lut[:, 1::2, :]
    plut = jnp.concatenate(
        [jnp.broadcast_to(even[:, None], (VOCAB, VOCAB, PSEQ, D)),
         jnp.broadcast_to(odd[None, :], (VOCAB, VOCAB, PSEQ, D))],
        axis=-1).reshape(VOCAB * VOCAB * PSEQ, 2 * D)
    xe = x[:, 0::2].reshape(PAIRS)
    xo = x[:, 1::2].reshape(PAIRS)
    out = _sc_gather(xe, xo, plut)
    return out.reshape(BATCH, SEQ, D)


# speedup vs baseline: 6.3741x; 1.0549x over previous
import functools

import jax
import jax.numpy as jnp
from jax import lax
from jax.experimental import pallas as pl
from jax.experimental.pallas import tpu as pltpu
from jax.experimental.pallas import tpu_sc as plsc

VOCAB = 4
SEQ = 10
D = 64
BATCH = 16384
ROWS = BATCH * SEQ
PAIRS = ROWS // 2
PSEQ = SEQ // 2

_info = plsc.get_sparse_core_info()
_NC = _info.num_cores
_NS = _info.num_subcores
_NW = _NC * _NS

PAIRS_PER_W = PAIRS // _NW
CHUNK = 320
N_CHUNKS = PAIRS_PER_W // CHUNK


def _lut_body(tok_ref, pos_ref, gamma_ref, beta_ref, lut_ref):
    tok = tok_ref[:, :]
    pos = pos_ref[:, :]
    e = tok[:, None, :] + pos[None, :, :]
    mean = jnp.mean(e, axis=-1, keepdims=True)
    c = e - mean
    var = jnp.mean(c * c, axis=-1, keepdims=True)
    normed = c * lax.rsqrt(var + 1e-5)
    lut_ref[:, :, :] = (normed * gamma_ref[0][None, None, :]
                        + beta_ref[0][None, None, :])


_lut_call = pl.pallas_call(
    _lut_body,
    out_shape=jax.ShapeDtypeStruct((VOCAB, SEQ, D), jnp.float32),
)


_sc_mesh = plsc.VectorSubcoreMesh(core_axis_name="c", subcore_axis_name="s")


@functools.partial(
    pl.kernel,
    mesh=_sc_mesh,
    out_type=jax.ShapeDtypeStruct((PAIRS, 2 * D), jnp.float32),
    scratch_types=[
        pltpu.VMEM((PAIRS_PER_W,), jnp.int32),
        pltpu.VMEM((PAIRS_PER_W,), jnp.int32),
        pltpu.VMEM((CHUNK, 2 * D), jnp.float32),
        pltpu.VMEM((CHUNK, 2 * D), jnp.float32),
        pltpu.SemaphoreType.DMA,
        pltpu.SemaphoreType.DMA,
        pltpu.SemaphoreType.DMA,
        pltpu.SemaphoreType.DMA,
    ],
)
def _sc_gather(c_hbm, plut_hbm, out_hbm, c_v, idx_v,
               rows0, rows1, gsem0, gsem1, osem0, osem1):
    wid = lax.axis_index("s") * _NC + lax.axis_index("c")
    base_w = wid * PAIRS_PER_W
    lane = lax.iota(jnp.int32, 16)

    pltpu.sync_copy(c_hbm.at[pl.ds(base_w, PAIRS_PER_W)], c_v)
    for g in range(PAIRS_PER_W // 16):
        off = g * 16
        p = lax.rem(base_w + off + lane, PSEQ)
        idx_v[pl.ds(off, 16)] = c_v[pl.ds(off, 16)] * PSEQ + p

    rows = (rows0, rows1)
    gsem = (gsem0, gsem1)
    osem = (osem0, osem1)
    gcp = [None, None]
    ocp = [None, None]
    for i in range(N_CHUNKS):
        b = i % 2
        if i >= 2:
            ocp[b].wait()
        gcp[b] = pltpu.make_async_copy(
            plut_hbm.at[idx_v.at[pl.ds(i * CHUNK, CHUNK)]], rows[b], gsem[b])
        gcp[b].start()
        if i >= 1:
            pb = (i - 1) % 2
            gcp[pb].wait()
            ocp[pb] = pltpu.make_async_copy(
                rows[pb], out_hbm.at[pl.ds(base_w + (i - 1) * CHUNK, CHUNK)],
                osem[pb])
            ocp[pb].start()
    lb = (N_CHUNKS - 1) % 2
    gcp[lb].wait()
    ocp[lb] = pltpu.make_async_copy(
        rows[lb], out_hbm.at[pl.ds(base_w + (N_CHUNKS - 1) * CHUNK, CHUNK)],
        osem[lb])
    ocp[lb].start()
    ocp[1 - lb].wait()
    ocp[lb].wait()


def kernel(x, tok_table, pos_table, gamma, beta):
    lut = _lut_call(tok_table, pos_table,
                    gamma.reshape(1, D), beta.reshape(1, D))
    even = lut[:, 0::2, :]
    odd = lut[:, 1::2, :]
    plut = jnp.concatenate(
        [jnp.broadcast_to(even[:, None], (VOCAB, VOCAB, PSEQ, D)),
         jnp.broadcast_to(odd[None, :], (VOCAB, VOCAB, PSEQ, D))],
        axis=-1).reshape(VOCAB * VOCAB * PSEQ, 2 * D)
    c = (x[:, 0::2] * VOCAB + x[:, 1::2]).reshape(PAIRS)
    out = _sc_gather(c, plut)
    return out.reshape(BATCH, SEQ, D)


# speedup vs baseline: 28.1889x; 4.4224x over previous
import functools

import jax
import jax.numpy as jnp
from jax import lax
from jax.experimental import pallas as pl
from jax.experimental.pallas import tpu as pltpu
from jax.experimental.pallas import tpu_sc as plsc

VOCAB = 4
SEQ = 10
D = 64
BATCH = 16384

_info = plsc.get_sparse_core_info()
_NC = _info.num_cores
_NS = _info.num_subcores
_NW = _NC * _NS
_L = 16

BW = BATCH // _NW
D8 = 8


def _lut_body(tok_ref, pos_ref, gamma_ref, beta_ref, lut_ref):
    tok = tok_ref[:, :]
    pos = pos_ref[:, :]
    e = tok[:, None, :] + pos[None, :, :]
    mean = jnp.mean(e, axis=-1, keepdims=True)
    c = e - mean
    var = jnp.mean(c * c, axis=-1, keepdims=True)
    normed = c * lax.rsqrt(var + 1e-5)
    lut_ref[:, :, :] = (normed * gamma_ref[0][None, None, :]
                        + beta_ref[0][None, None, :])


_lut_call = pl.pallas_call(
    _lut_body,
    out_shape=jax.ShapeDtypeStruct((VOCAB, SEQ, D), jnp.float32),
)


_sc_mesh = plsc.VectorSubcoreMesh(core_axis_name="c", subcore_axis_name="s")


@functools.partial(
    pl.kernel,
    mesh=_sc_mesh,
    out_type=jax.ShapeDtypeStruct((SEQ, D, BATCH), jnp.float32),
    scratch_types=[
        pltpu.VMEM((SEQ * D * VOCAB * _L // 128, 128), jnp.float32),
        pltpu.VMEM((SEQ, BW), jnp.int32),
        pltpu.VMEM((D, BW), jnp.float32),
        pltpu.VMEM((D, BW), jnp.float32),
        pltpu.SemaphoreType.DMA,
        pltpu.SemaphoreType.DMA,
    ],
)
def _sc_select(rep_hbm, xt_hbm, out_hbm, rep_v, x_v, slab0, slab1,
               osem0, osem1):
    wid = lax.axis_index("s") * _NC + lax.axis_index("c")
    b0 = wid * BW

    pltpu.sync_copy(rep_hbm, rep_v)
    for s in range(SEQ):
        pltpu.sync_copy(xt_hbm.at[pl.ds(s * BATCH + b0, BW)], x_v.at[s])

    slabs = (slab0, slab1)
    osem = (osem0, osem1)
    ocp = [None, None]
    for s in range(SEQ):
        sb = s % 2
        if ocp[sb] is not None:
            ocp[sb].wait()
        slab = slabs[sb]
        for dblk in range(D // D8):
            A = []
            for j in range(D8):
                flat = ((s * D + dblk * D8 + j) * VOCAB) * _L
                A.append([rep_v[(flat + v * _L) // 128,
                                pl.ds((flat + v * _L) % 128, _L)]
                          for v in range(VOCAB)])

            def body(b16, carry, s=s, dblk=dblk, A=A, slab=slab):
                off = b16 * _L
                c = x_v[s, pl.ds(off, _L)]
                m1 = c == 1
                m2 = c == 2
                m3 = c == 3
                for j in range(D8):
                    r = jnp.where(m1, A[j][1], A[j][0])
                    r = jnp.where(m2, A[j][2], r)
                    r = jnp.where(m3, A[j][3], r)
                    slab[dblk * D8 + j, pl.ds(off, _L)] = r
                return carry

            lax.fori_loop(0, BW // _L, body, 0)
        ocp[sb] = pltpu.make_async_copy(
            slab, out_hbm.at[s, :, pl.ds(b0, BW)], osem[sb])
        ocp[sb].start()
    ocp[0].wait()
    ocp[1].wait()


def kernel(x, tok_table, pos_table, gamma, beta):
    lut = _lut_call(tok_table, pos_table,
                    gamma.reshape(1, D), beta.reshape(1, D))
    rep = jnp.broadcast_to(
        lut.transpose(1, 2, 0)[:, :, :, None], (SEQ, D, VOCAB, _L)
    ).reshape(SEQ * D * VOCAB * _L // 128, 128)
    xt = x.T.reshape(SEQ * BATCH)
    out = _sc_select(rep, xt)
    return jnp.transpose(out, (2, 0, 1))
